# Initial kernel scaffold; baseline (speedup 1.0000x reference)
import jax
import jax.numpy as jnp
from jax.experimental import pallas as pl


def kernel(x, node_weights, edge_weights, lin_W, lin_b, batch, edge_index, v, lin, W_rff, b_rff):
    raise NotImplementedError("write your pallas kernel here")


# speedup vs baseline: 24.5760x; 24.5760x over previous
import functools
import math

import jax
import jax.numpy as jnp
from jax import lax
from jax.experimental import pallas as pl
from jax.experimental.pallas import tpu as pltpu
from jax.experimental.pallas import tpu_sc as plsc

_N = 10000
_E = 160000
_G = 64
_DIM = 128
_D = 16
_S = 16
_RFF = 4096
_C = 10
_SCALE = 100.0

_NP = 10240
_EP = 163840
_TN = 1024
_TE = 1024
_NT_N = _NP // _TN
_NT_E = _EP // _TE
_SD = _S * _D
_HI = lax.Precision.HIGHEST


def _node_kernel(x_ref, nw_ref, b_ref, v_ref, lin_ref, nh_ref, accp_ref, acc):
    i = pl.program_id(0)

    @pl.when(i == 0)
    def _init():
        acc[...] = jnp.zeros_like(acc)

    xw = x_ref[...] * nw_ref[...]
    nh = jnp.dot(xw.astype(jnp.bfloat16), v_ref[...],
                 preferred_element_type=jnp.float32)
    nh_ref[...] = nh
    gids = b_ref[0]
    gi = lax.broadcasted_iota(jnp.int32, (_G, _TN), 0)
    oh = (gids == gi).astype(jnp.float32)
    rep = jnp.tile(nh[:, 0:_D], (1, _S))
    bump = 0.5 * jnp.tanh(0.5 * (_SCALE * (lin_ref[...] - rep))) + 0.5
    acc[...] += lax.dot_general(oh, bump, (((1,), (0,)), ((), ())),
                                preferred_element_type=jnp.float32,
                                precision=_HI)

    @pl.when(i == _NT_N - 1)
    def _fin():
        accp_ref[...] = acc[...]


def _edge_kernel(rs_ref, rd_ref, src_ref, ew_ref, br_ref, lin_ref, accp_ref,
                 wrff_ref, brff_ref, lw_ref, lb_ref, logits_ref, flat_ref,
                 acc, st):
    i = pl.program_id(0)

    @pl.when(i == 0)
    def _init():
        acc[...] = jnp.zeros_like(acc)
        br = br_ref[...]
        gi = lax.broadcasted_iota(jnp.int32, (_G, _NP), 0)
        lo = jnp.sum((br < gi).astype(jnp.float32), axis=1, keepdims=True)
        hi = jnp.sum((br <= gi).astype(jnp.float32), axis=1, keepdims=True)
        st[:, 0:1] = lo
        st[:, 1:2] = hi

    @pl.when(i < _NT_E)
    def _acc():
        src_f = src_ref[0].astype(jnp.float32)
        lo = st[:, 0:1]
        hi = st[:, 1:2]
        col = lax.broadcasted_iota(jnp.int32, (1, _TE), 1)
        vm = (i * _TE + col) < _E
        oh = ((src_f >= lo) & (src_f < hi) & vm).astype(jnp.float32)
        eh = jnp.maximum(rs_ref[...], rd_ref[...]) * ew_ref[0]
        rep = jnp.tile(eh, (1, _S))
        bump = 0.5 * jnp.tanh(0.5 * (_SCALE * (lin_ref[...] - rep))) + 0.5
        acc[...] += lax.dot_general(oh, bump, (((1,), (0,)), ((), ())),
                                    preferred_element_type=jnp.float32,
                                    precision=_HI)

    @pl.when(i == _NT_E)
    def _fin():
        flat = accp_ref[...] - acc[...]
        flat_ref[...] = flat
        z = jnp.dot(flat.astype(jnp.bfloat16), wrff_ref[...],
                    preferred_element_type=jnp.float32) + brff_ref[...]
        phi = math.sqrt(2.0 / _RFF) * jnp.cos(z)
        logits_ref[...] = jnp.dot(phi.astype(jnp.bfloat16), lw_ref[...],
                                  preferred_element_type=jnp.float32) + lb_ref[...]


def _sc_gather(nh_wide, src_flat, dst_flat):
    info = plsc.get_sparse_core_info()
    nw = info.num_cores * info.num_subcores
    pw = _EP // nw
    ch = 64
    nch = pw // ch
    orow = ch * _D // 128
    mesh = plsc.VectorSubcoreMesh(core_axis_name="c", subcore_axis_name="s")

    @functools.partial(
        pl.kernel,
        out_type=(jax.ShapeDtypeStruct((_EP * _D // 128, 128), jnp.float32),
                  jax.ShapeDtypeStruct((_EP * _D // 128, 128), jnp.float32)),
        mesh=mesh,
        scratch_types=[
            pltpu.VMEM_SHARED((_NP, 128), jnp.float32),
            pltpu.VMEM((ch,), jnp.int32),
            pltpu.VMEM((ch,), jnp.int32),
            pltpu.VMEM((ch, 128), jnp.float32),
            pltpu.VMEM((ch, 128), jnp.float32),
            pltpu.VMEM((orow, 128), jnp.float32),
            pltpu.VMEM((orow, 128), jnp.float32),
            pltpu.SemaphoreType.DMA,
            pltpu.SemaphoreType.DMA,
        ],
    )
    def k(nh_hbm, s_hbm, d_hbm, os_hbm, od_hbm,
          nh_sh, sidx, didx, srows, drows, sout, dout, sem_s, sem_d):
        sid = lax.axis_index("s")
        wid = sid * info.num_cores + lax.axis_index("c")
        base = wid * pw

        @pl.when(sid == 0)
        def _stage():
            pltpu.sync_copy(nh_hbm, nh_sh)

        plsc.subcore_barrier()

        def body(j, carry):
            off = pl.multiple_of(base + j * ch, ch)
            pltpu.sync_copy(s_hbm.at[pl.ds(off, ch)], sidx)
            pltpu.sync_copy(d_hbm.at[pl.ds(off, ch)], didx)
            pltpu.sync_copy(nh_sh.at[sidx], srows)
            pltpu.sync_copy(nh_sh.at[didx], drows)
            for r in range(orow):
                for b in range(128 // _D):
                    e = r * (128 // _D) + b
                    sout[r, pl.ds(b * _D, _D)] = srows[e, 0:_D]
                    dout[r, pl.ds(b * _D, _D)] = drows[e, 0:_D]
            obase = pl.multiple_of(off * _D // 128, orow)
            pltpu.sync_copy(sout, os_hbm.at[pl.ds(obase, orow)])
            pltpu.sync_copy(dout, od_hbm.at[pl.ds(obase, orow)])
            return carry

        lax.fori_loop(0, nch, body, 0)

    return k(nh_wide, src_flat, dst_flat)


def kernel(x, node_weights, edge_weights, lin_W, lin_b, batch, edge_index, v,
           lin, W_rff, b_rff):
    f32 = jnp.float32
    x_p = jnp.pad(x, ((0, _NP - _N), (0, 0)))
    nw_p = jnp.pad(node_weights, (0, _NP - _N)).reshape(_NP, 1)
    b_pad = jnp.pad(batch.astype(jnp.int32), (0, _NP - _N),
                    constant_values=_G)
    b2d = b_pad.reshape(_NT_N, 1, _TN)
    b_row = b_pad.reshape(1, _NP)
    src_p = jnp.pad(edge_index[0].astype(jnp.int32), (0, _EP - _E))
    dst_p = jnp.pad(edge_index[1].astype(jnp.int32), (0, _EP - _E))
    src2d = src_p.reshape(_NT_E, 1, _TE)
    ew3d = jnp.pad(edge_weights, (0, _EP - _E)).reshape(_NT_E, _TE, 1)
    lin_row = jnp.repeat(lin.reshape(_S), _D).reshape(1, _SD)
    brff_row = b_rff.reshape(1, _RFF)
    lw_pad = jnp.pad(lin_W, ((0, 0), (0, 128 - _C)))
    lb_pad = jnp.pad(lin_b, (0, 128 - _C)).reshape(1, 128)

    nh_pad, acc_pts = pl.pallas_call(
        _node_kernel,
        grid=(_NT_N,),
        in_specs=[
            pl.BlockSpec((_TN, _DIM), lambda i: (i, 0)),
            pl.BlockSpec((_TN, 1), lambda i: (i, 0)),
            pl.BlockSpec((1, 1, _TN), lambda i: (i, 0, 0)),
            pl.BlockSpec((_DIM, 128), lambda i: (0, 0)),
            pl.BlockSpec((1, _SD), lambda i: (0, 0)),
        ],
        out_specs=[
            pl.BlockSpec((_TN, 128), lambda i: (i, 0)),
            pl.BlockSpec((_G, _SD), lambda i: (0, 0)),
        ],
        out_shape=[
            jax.ShapeDtypeStruct((_NP, 128), f32),
            jax.ShapeDtypeStruct((_G, _SD), f32),
        ],
        scratch_shapes=[pltpu.VMEM((_G, _SD), f32)],
    )(x_p, nw_p, b2d,
      jnp.pad(v.astype(jnp.bfloat16), ((0, 0), (0, 128 - _D))), lin_row)

    rows_s1, rows_d1 = _sc_gather(nh_pad, src_p, dst_p)
    rows_s = rows_s1.reshape(_EP, _D)
    rows_d = rows_d1.reshape(_EP, _D)

    clamp = _NT_E - 1
    logits_pad, flat = pl.pallas_call(
        _edge_kernel,
        grid=(_NT_E + 1,),
        in_specs=[
            pl.BlockSpec((_TE, _D), lambda i: (jnp.minimum(i, clamp), 0)),
            pl.BlockSpec((_TE, _D), lambda i: (jnp.minimum(i, clamp), 0)),
            pl.BlockSpec((1, 1, _TE), lambda i: (jnp.minimum(i, clamp), 0, 0)),
            pl.BlockSpec((1, _TE, 1), lambda i: (jnp.minimum(i, clamp), 0, 0)),
            pl.BlockSpec((1, _NP), lambda i: (0, 0)),
            pl.BlockSpec((1, _SD), lambda i: (0, 0)),
            pl.BlockSpec((_G, _SD), lambda i: (0, 0)),
            pl.BlockSpec((_SD, _RFF), lambda i: (0, 0)),
            pl.BlockSpec((1, _RFF), lambda i: (0, 0)),
            pl.BlockSpec((_RFF, 128), lambda i: (0, 0)),
            pl.BlockSpec((1, 128), lambda i: (0, 0)),
        ],
        out_specs=[
            pl.BlockSpec((_G, 128), lambda i: (0, 0)),
            pl.BlockSpec((_G, _SD), lambda i: (0, 0)),
        ],
        out_shape=[
            jax.ShapeDtypeStruct((_G, 128), f32),
            jax.ShapeDtypeStruct((_G, _SD), f32),
        ],
        scratch_shapes=[
            pltpu.VMEM((_G, _SD), f32),
            pltpu.VMEM((_G, 128), f32),
        ],
    )(rows_s, rows_d, src2d, ew3d, b_row, lin_row, acc_pts,
      W_rff.astype(jnp.bfloat16), brff_row, lw_pad.astype(jnp.bfloat16),
      lb_pad)

    return (logits_pad[:, :_C], flat)
